# Initial kernel scaffold; baseline (speedup 1.0000x reference)
import jax
import jax.numpy as jnp
from jax.experimental import pallas as pl


def kernel(x, ln_gamma, ln_beta, router_W, router_b, W1, b1, W2, b2):
    raise NotImplementedError("write your pallas kernel here")


# speedup vs baseline: 2.6543x; 2.6543x over previous
import jax
import jax.numpy as jnp
from jax.experimental import pallas as pl
from jax.experimental.pallas import tpu as pltpu

_T, _D, _H, _E = 2048, 768, 1536, 8
_EPS = 1e-5
_BT = 256
_NT = _T // _BT


def _moe_dense_kernel(x_ref, g_ref, bta_ref, rw_ref, rb_ref,
                      w1_ref, b1_ref, w2_ref, b2_ref,
                      y_ref,
                      xn_ref, wf_ref, w1s_ref, w2s_ref):
    e = pl.program_id(0)
    t = pl.program_id(1)

    @pl.when((e == 0) & (t == 0))
    def _prologue():
        xx = x_ref[...]
        mu = jnp.mean(xx, axis=1, keepdims=True)
        xc = xx - mu
        var = jnp.mean(xc * xc, axis=1, keepdims=True)
        xn = xc * jax.lax.rsqrt(var + _EPS)
        xn = xn * g_ref[...] + bta_ref[...]
        xn_ref[...] = xn.astype(jnp.bfloat16)
        logits = jnp.dot(xn, rw_ref[...],
                         preferred_element_type=jnp.float32) + rb_ref[...]
        ids = jax.lax.broadcasted_iota(jnp.int32, (_T, _E), 1)
        m1 = jnp.max(logits, axis=1, keepdims=True)
        i1 = jnp.min(jnp.where(logits == m1, ids, _E), axis=1, keepdims=True)
        l2 = jnp.where(ids == i1, -jnp.inf, logits)
        m2 = jnp.max(l2, axis=1, keepdims=True)
        i2 = jnp.min(jnp.where(l2 == m2, ids, _E), axis=1, keepdims=True)
        e2 = jnp.exp(m2 - m1)
        denom = 1.0 + e2
        wf_ref[...] = (jnp.where(ids == i1, 1.0, 0.0)
                       + jnp.where(ids == i2, e2, 0.0)) / denom

    @pl.when(t == 0)
    def _cast_weights():
        w1s_ref[...] = w1_ref[0].astype(jnp.bfloat16)
        w2s_ref[...] = w2_ref[0].astype(jnp.bfloat16)

    xt = xn_ref[pl.ds(t * _BT, _BT), :]
    h = jnp.dot(xt, w1s_ref[...], preferred_element_type=jnp.float32)
    h = h + b1_ref[0]
    h = 0.5 * h * (1.0 + jax.lax.erf(h * 0.7071067811865476))
    yb = jnp.dot(h.astype(jnp.bfloat16), w2s_ref[...],
                 preferred_element_type=jnp.float32)
    yb = yb + b2_ref[0]
    wtile = wf_ref[pl.ds(t * _BT, _BT), :]
    ids8 = jax.lax.broadcasted_iota(jnp.int32, (_BT, _E), 1)
    w = jnp.sum(jnp.where(ids8 == e, wtile, 0.0), axis=1, keepdims=True)
    contrib = yb * w

    @pl.when(e == 0)
    def _init():
        y_ref[pl.ds(t * _BT, _BT), :] = contrib

    @pl.when(e > 0)
    def _acc():
        y_ref[pl.ds(t * _BT, _BT), :] += contrib


def kernel(x, ln_gamma, ln_beta, router_W, router_b, W1, b1, W2, b2):
    g2 = ln_gamma.reshape(1, _D)
    bta2 = ln_beta.reshape(1, _D)
    rb2 = router_b.reshape(1, _E)

    b1r = b1.reshape(_E, 1, _H)
    b2r = b2.reshape(_E, 1, _D)

    full = lambda shape: pl.BlockSpec(shape, lambda e, t: (0,) * len(shape))
    per_e3 = lambda d1, d2: pl.BlockSpec((1, d1, d2), lambda e, t: (e, 0, 0))

    return pl.pallas_call(
        _moe_dense_kernel,
        grid=(_E, _NT),
        in_specs=[
            full((_T, _D)),
            full((1, _D)),
            full((1, _D)),
            full((_D, _E)),
            full((1, _E)),
            per_e3(_D, _H),
            per_e3(1, _H),
            per_e3(_H, _D),
            per_e3(1, _D),
        ],
        out_specs=full((_T, _D)),
        out_shape=jax.ShapeDtypeStruct((_T, _D), jnp.float32),
        scratch_shapes=[
            pltpu.VMEM((_T, _D), jnp.bfloat16),
            pltpu.VMEM((_T, _E), jnp.float32),
            pltpu.VMEM((_D, _H), jnp.bfloat16),
            pltpu.VMEM((_H, _D), jnp.bfloat16),
        ],
        compiler_params=pltpu.CompilerParams(
            dimension_semantics=("arbitrary", "arbitrary"),
        ),
    )(x, g2, bta2, router_W, rb2, W1, b1r, W2, b2r)
